# baseline (speedup 1.0000x reference)
import functools

import jax
import jax.numpy as jnp
from jax import lax
from jax.experimental import pallas as pl
from jax.experimental.pallas import tpu as pltpu
from jax.experimental.pallas import tpu_sc as plsc

N = 10000
E = 320000
NC = 2
NS = 16
K = 80
RPT = 640
RPT_LAST = N - 15 * RPT
EPT_AGG = E // NS
EPT_DEG = E // (NC * NS)
ROW_BLK = 1000

_sc_mesh = plsc.VectorSubcoreMesh(core_axis_name="c", subcore_axis_name="s")


def _deg_body(dst_hbm, ones_hbm, degp_hbm, ones_v, didx0, didx1,
              isem0, isem1, ssem0, ssem1, deg_sp):
    c = lax.axis_index("c")
    s = lax.axis_index("s")
    didx = (didx0, didx1)
    isem = (isem0, isem1)
    ssem = (ssem0, ssem1)
    ncha = EPT_DEG // K

    def dchunk(j):
        return dst_hbm.at[pl.ds((c * NS + s) * EPT_DEG + j * K, K)]

    pltpu.sync_copy(ones_hbm.at[pl.ds(0, K)], ones_v)

    @pl.when(s < 15)
    def _():
        pltpu.sync_copy(ones_hbm.at[pl.ds(s * RPT, RPT)],
                        deg_sp.at[pl.ds(s * RPT, RPT)])

    @pl.when(s == 15)
    def _():
        pltpu.sync_copy(ones_hbm.at[pl.ds(15 * RPT, RPT_LAST)],
                        deg_sp.at[pl.ds(15 * RPT, RPT_LAST)])

    plsc.subcore_barrier()
    pltpu.async_copy(dchunk(0), didx0, isem0)

    def step(o, _):
        for b in (0, 1):
            jl = 2 * o + b
            pltpu.make_async_copy(dchunk(jl), didx[b], isem[b]).wait()
            pltpu.async_copy(ones_v, deg_sp.at[didx[b]], ssem[b], add=True)
            if b == 0:
                @pl.when(o > 0)
                def _():
                    pltpu.make_async_copy(ones_v, deg_sp.at[didx1],
                                          ssem1).wait()
            else:
                pltpu.make_async_copy(ones_v, deg_sp.at[didx0], ssem0).wait()
            pltpu.async_copy(dchunk(jl + 1), didx[1 - b], isem[1 - b])
        return 0

    lax.fori_loop(0, ncha // 2, step, 0)
    pltpu.make_async_copy(dchunk(ncha - 1), didx0, isem0).wait()
    pltpu.make_async_copy(ones_v, deg_sp.at[didx1], ssem1).wait()
    pltpu.sync_copy(ones_v, deg_sp.at[didx0], add=True)
    plsc.subcore_barrier()

    @pl.when(s < 15)
    def _():
        pltpu.sync_copy(deg_sp.at[pl.ds(s * RPT, RPT)],
                        degp_hbm.at[c, pl.ds(s * RPT, RPT)])

    @pl.when(s == 15)
    def _():
        pltpu.sync_copy(deg_sp.at[pl.ds(15 * RPT, RPT_LAST)],
                        degp_hbm.at[c, pl.ds(15 * RPT, RPT_LAST)])


_deg_call = pl.kernel(
    _deg_body,
    out_type=jax.ShapeDtypeStruct((NC, N, 16), jnp.float32),
    mesh=_sc_mesh,
    scratch_types=[
        pltpu.VMEM((K, 16), jnp.float32),
        pltpu.VMEM((K,), jnp.int32),
        pltpu.VMEM((K,), jnp.int32),
        pltpu.SemaphoreType.DMA,
        pltpu.SemaphoreType.DMA,
        pltpu.SemaphoreType.DMA,
        pltpu.SemaphoreType.DMA,
        pltpu.VMEM_SHARED((N, 16), jnp.float32),
    ],
)


NCHUNK = EPT_AGG // K
R = 4
NGRP = (NCHUNK - 2) // R


def _agg_body(y_hbm, srcoff_hbm, dst_hbm, out_hbm, *scratch):
    c = lax.axis_index("c")
    s = lax.axis_index("s")
    sidx = scratch[0:R]
    didx = scratch[R:2 * R]
    rows = scratch[2 * R:3 * R]
    jsem = scratch[3 * R:4 * R]
    isem = scratch[4 * R:5 * R]
    gsem = scratch[5 * R:6 * R]
    ssem = scratch[6 * R:7 * R]
    acc_sp = scratch[7 * R]

    def schunk_hbm(j):
        return srcoff_hbm.at[
            pl.ds(pl.multiple_of(c * E + s * EPT_AGG + j * K, 8), K)]

    def dchunk_hbm(j):
        return dst_hbm.at[pl.ds(s * EPT_AGG + j * K, K)]

    @pl.when(s < 15)
    def _():
        start = pl.multiple_of(c * N + s * RPT, RPT)
        pltpu.sync_copy(y_hbm.at[pl.ds(start, RPT)],
                        acc_sp.at[pl.ds(s * RPT, RPT)])

    @pl.when(s == 15)
    def _():
        start = pl.multiple_of(c * N + 15 * RPT, 16)
        pltpu.sync_copy(y_hbm.at[pl.ds(start, RPT_LAST)],
                        acc_sp.at[pl.ds(15 * RPT, RPT_LAST)])

    for k in range(R):
        pltpu.async_copy(schunk_hbm(k), sidx[k], jsem[k])
    for k in range(R - 1):
        pltpu.async_copy(dchunk_hbm(k), didx[k], isem[k])
    for k in range(R - 1):
        pltpu.make_async_copy(schunk_hbm(k), sidx[k], jsem[k]).wait()
        pltpu.async_copy(y_hbm.at[sidx[k]], rows[k], gsem[k])

    def step(t, _):
        for k in range(R):
            j = R * t + k
            kp = (k + R - 1) % R
            pltpu.make_async_copy(y_hbm.at[sidx[k]], rows[k], gsem[k]).wait()

            def prefetch_sidx(j=j, k=k):
                pltpu.async_copy(schunk_hbm(j + R), sidx[k], jsem[k])

            if k < 2:
                prefetch_sidx()
            else:
                pl.when(t < NGRP - 1)(prefetch_sidx)
            pltpu.make_async_copy(dchunk_hbm(j), didx[k], isem[k]).wait()
            pltpu.async_copy(rows[k], acc_sp.at[didx[k]], ssem[k], add=True)
            if k == 0:
                @pl.when(t > 0)
                def _():
                    pltpu.make_async_copy(rows[R - 1], acc_sp.at[didx[R - 1]],
                                          ssem[R - 1]).wait()
            else:
                pltpu.make_async_copy(rows[kp], acc_sp.at[didx[kp]],
                                      ssem[kp]).wait()

            def issue_next(j=j, kp=kp):
                pltpu.async_copy(dchunk_hbm(j + R - 1), didx[kp], isem[kp])
                pltpu.make_async_copy(schunk_hbm(j + R - 1), sidx[kp],
                                      jsem[kp]).wait()
                pltpu.async_copy(y_hbm.at[sidx[kp]], rows[kp], gsem[kp])

            if k < R - 1:
                issue_next()
            else:
                pl.when(t < NGRP - 1)(issue_next)
        return 0

    lax.fori_loop(0, NGRP, step, 0)
    pltpu.make_async_copy(y_hbm.at[sidx[0]], rows[0], gsem[0]).wait()
    pltpu.make_async_copy(dchunk_hbm(NCHUNK - 2), didx[0], isem[0]).wait()
    pltpu.async_copy(rows[0], acc_sp.at[didx[0]], ssem[0], add=True)
    pltpu.make_async_copy(rows[R - 1], acc_sp.at[didx[R - 1]],
                          ssem[R - 1]).wait()
    pltpu.make_async_copy(y_hbm.at[sidx[1]], rows[1], gsem[1]).wait()
    pltpu.make_async_copy(dchunk_hbm(NCHUNK - 1), didx[1], isem[1]).wait()
    pltpu.sync_copy(rows[1], acc_sp.at[didx[1]], add=True)
    pltpu.make_async_copy(rows[0], acc_sp.at[didx[0]], ssem[0]).wait()
    plsc.subcore_barrier()

    @pl.when(s < 15)
    def _():
        pltpu.sync_copy(acc_sp.at[pl.ds(s * RPT, RPT)],
                        out_hbm.at[c, pl.ds(s * RPT, RPT)])

    @pl.when(s == 15)
    def _():
        pltpu.sync_copy(acc_sp.at[pl.ds(15 * RPT, RPT_LAST)],
                        out_hbm.at[c, pl.ds(15 * RPT, RPT_LAST)])


_agg_call = pl.kernel(
    _agg_body,
    out_type=jax.ShapeDtypeStruct((NC, N, 128), jnp.float32),
    mesh=_sc_mesh,
    scratch_types=(
        [pltpu.VMEM((K,), jnp.int32)] * (2 * R)
        + [pltpu.VMEM((K, 128), jnp.float32)] * R
        + [pltpu.SemaphoreType.DMA] * (4 * R)
        + [pltpu.VMEM_SHARED((N, 128), jnp.float32)]
    ),
)


def _dinv(degp_ref):
    deg = degp_ref[0, :, 0:1] + degp_ref[1, :, 0:1] - 1.0
    return lax.rsqrt(deg)


def _split_out(y_ref, y):
    y_ref[0] = y[:, :128]
    y_ref[1] = y[:, 128:]


def _layer0_body(x_ref, degp_ref, w_ref, y_ref):
    y = _dinv(degp_ref) * jnp.dot(x_ref[...], w_ref[...],
                                  preferred_element_type=jnp.float32)
    _split_out(y_ref, y)


def _layer_body(acc_ref, degp_ref, b_ref, w_ref, y_ref):
    dinv = _dinv(degp_ref)
    acc = jnp.concatenate([acc_ref[0], acc_ref[1]], axis=1)
    h = jax.nn.relu(dinv * acc + b_ref[...])
    y = dinv * jnp.dot(h, w_ref[...], preferred_element_type=jnp.float32)
    _split_out(y_ref, y)


def _final_body(acc_ref, degp_ref, b_ref, wf1_ref, bf1_ref, wf2_ref, bf2_ref, o_ref):
    dinv = _dinv(degp_ref)
    acc = jnp.concatenate([acc_ref[0], acc_ref[1]], axis=1)
    h = jax.nn.relu(dinv * acc + b_ref[...])
    t = jnp.dot(h, wf1_ref[...], preferred_element_type=jnp.float32) + bf1_ref[...]
    o = jnp.dot(t, wf2_ref[...], preferred_element_type=jnp.float32) + bf2_ref[...]
    m = jnp.max(o, axis=1, keepdims=True)
    sh = o - m
    o_ref[...] = sh - jnp.log(jnp.sum(jnp.exp(sh), axis=1, keepdims=True))


def _row_spec(d):
    return pl.BlockSpec((ROW_BLK, d), lambda i: (i, 0))


def _split_spec(d):
    return pl.BlockSpec((NC, ROW_BLK, d), lambda i: (0, i, 0))


def _full_spec(a, b):
    return pl.BlockSpec((a, b), lambda i: (0, 0))


_GRID = (N // ROW_BLK,)


def _layer0(x, degp, W):
    return pl.pallas_call(
        _layer0_body,
        grid=_GRID,
        in_specs=[_row_spec(128), _split_spec(16), _full_spec(128, 256)],
        out_specs=_split_spec(128),
        out_shape=jax.ShapeDtypeStruct((NC, N, 128), jnp.float32),
    )(x, degp, W)


def _layer(acc, degp, b, W):
    return pl.pallas_call(
        _layer_body,
        grid=_GRID,
        in_specs=[_split_spec(128), _split_spec(16), _full_spec(1, 256),
                  _full_spec(256, 256)],
        out_specs=_split_spec(128),
        out_shape=jax.ShapeDtypeStruct((NC, N, 128), jnp.float32),
    )(acc, degp, b.reshape(1, -1), W)


def _final(acc, degp, b, Wf1, bf1, Wf2, bf2):
    return pl.pallas_call(
        _final_body,
        grid=_GRID,
        in_specs=[_split_spec(128), _split_spec(16), _full_spec(1, 256),
                  _full_spec(256, 256), _full_spec(1, 256),
                  _full_spec(256, 128), _full_spec(1, 128)],
        out_specs=_row_spec(128),
        out_shape=jax.ShapeDtypeStruct((N, 128), jnp.float32),
    )(acc, degp, b.reshape(1, -1), Wf1, bf1.reshape(1, -1), Wf2, bf2.reshape(1, -1))


def kernel(x, edge_index, W0, b0, W1, b1, W2, b2, Wf1, bf1, Wf2, bf2):
    src = edge_index[0].astype(jnp.int32)
    dst = edge_index[1].astype(jnp.int32)
    degp = _deg_call(dst, jnp.ones((N, 16), jnp.float32))
    srcoff = jnp.concatenate([src, src + N])

    def agg(y):
        return _agg_call(y.reshape(NC * N, 128), srcoff, dst)

    y = _layer0(x, degp, W0)
    acc = agg(y)
    y = _layer(acc, degp, b0, W1)
    acc = agg(y)
    y = _layer(acc, degp, b1, W2)
    acc = agg(y)
    return _final(acc, degp, b2, Wf1, bf1, Wf2, bf2)


# speedup vs baseline: 1.0589x; 1.0589x over previous
import functools

import jax
import jax.numpy as jnp
from jax import lax
from jax.experimental import pallas as pl
from jax.experimental.pallas import tpu as pltpu
from jax.experimental.pallas import tpu_sc as plsc

N = 10000
E = 320000
NC = 2
NS = 16
K = 80
RPT = 640
RPT_LAST = N - 15 * RPT
EPT_AGG = E // NS
EPT_DEG = E // (NC * NS)
ROW_BLK = 1000

_sc_mesh = plsc.VectorSubcoreMesh(core_axis_name="c", subcore_axis_name="s")


def _deg_body(dst_hbm, ones_hbm, degp_hbm, ones_v, didx0, didx1, didx2,
              isem0, isem1, isem2, ssem0, ssem1, ssem2, deg_sp):
    c = lax.axis_index("c")
    s = lax.axis_index("s")
    didx = (didx0, didx1, didx2)
    isem = (isem0, isem1, isem2)
    ssem = (ssem0, ssem1, ssem2)
    ncha = EPT_DEG // K

    def dchunk(j):
        return dst_hbm.at[pl.ds((c * NS + s) * EPT_DEG + j * K, K)]

    pltpu.sync_copy(ones_hbm.at[pl.ds(0, K)], ones_v)

    @pl.when(s < 15)
    def _():
        pltpu.sync_copy(ones_hbm.at[pl.ds(s * RPT, RPT)],
                        deg_sp.at[pl.ds(s * RPT, RPT)])

    @pl.when(s == 15)
    def _():
        pltpu.sync_copy(ones_hbm.at[pl.ds(15 * RPT, RPT_LAST)],
                        deg_sp.at[pl.ds(15 * RPT, RPT_LAST)])

    plsc.subcore_barrier()
    pltpu.async_copy(dchunk(0), didx0, isem0)
    pltpu.async_copy(dchunk(1), didx1, isem1)

    def step(t, _):
        for k in (0, 1, 2):
            j = 3 * t + k
            kp = (k + 2) % 3
            pltpu.make_async_copy(dchunk(j), didx[k], isem[k]).wait()
            pltpu.async_copy(ones_v, deg_sp.at[didx[k]], ssem[k], add=True)
            if k == 0:
                @pl.when(t > 0)
                def _():
                    pltpu.make_async_copy(ones_v, deg_sp.at[didx2],
                                          ssem2).wait()
            else:
                pltpu.make_async_copy(ones_v, deg_sp.at[didx[kp]],
                                      ssem[kp]).wait()
            pltpu.async_copy(dchunk(j + 2), didx[kp], isem[kp])
        return 0

    lax.fori_loop(0, (ncha - 2) // 3, step, 0)
    pltpu.make_async_copy(dchunk(ncha - 2), didx0, isem0).wait()
    pltpu.async_copy(ones_v, deg_sp.at[didx0], ssem0, add=True)
    pltpu.make_async_copy(ones_v, deg_sp.at[didx2], ssem2).wait()
    pltpu.make_async_copy(dchunk(ncha - 1), didx1, isem1).wait()
    pltpu.sync_copy(ones_v, deg_sp.at[didx1], add=True)
    pltpu.make_async_copy(ones_v, deg_sp.at[didx0], ssem0).wait()
    plsc.subcore_barrier()

    @pl.when(s < 15)
    def _():
        pltpu.sync_copy(deg_sp.at[pl.ds(s * RPT, RPT)],
                        degp_hbm.at[c, pl.ds(s * RPT, RPT)])

    @pl.when(s == 15)
    def _():
        pltpu.sync_copy(deg_sp.at[pl.ds(15 * RPT, RPT_LAST)],
                        degp_hbm.at[c, pl.ds(15 * RPT, RPT_LAST)])


_deg_call = pl.kernel(
    _deg_body,
    out_type=jax.ShapeDtypeStruct((NC, N, 16), jnp.float32),
    mesh=_sc_mesh,
    scratch_types=(
        [pltpu.VMEM((K, 16), jnp.float32)]
        + [pltpu.VMEM((K,), jnp.int32)] * 3
        + [pltpu.SemaphoreType.DMA] * 6
        + [pltpu.VMEM_SHARED((N, 16), jnp.float32)]
    ),
)


NCHUNK = EPT_AGG // K
NTRI = (NCHUNK - 1) // 3


def _agg_body(y_hbm, srcoff_hbm, dst_hbm, out_hbm,
              sidx0, sidx1, sidx2, didx0, didx1, didx2, rows0, rows1, rows2,
              jsem0, jsem1, jsem2, isem0, isem1, isem2,
              gsem0, gsem1, gsem2, ssem0, ssem1, ssem2,
              acc_sp):
    c = lax.axis_index("c")
    s = lax.axis_index("s")
    sidx = (sidx0, sidx1, sidx2)
    didx = (didx0, didx1, didx2)
    rows = (rows0, rows1, rows2)
    jsem = (jsem0, jsem1, jsem2)
    isem = (isem0, isem1, isem2)
    gsem = (gsem0, gsem1, gsem2)
    ssem = (ssem0, ssem1, ssem2)

    def schunk_hbm(j):
        return srcoff_hbm.at[
            pl.ds(pl.multiple_of(c * E + s * EPT_AGG + j * K, 8), K)]

    def dchunk_hbm(j):
        return dst_hbm.at[pl.ds(s * EPT_AGG + j * K, K)]

    @pl.when(s < 15)
    def _():
        start = pl.multiple_of(c * N + s * RPT, RPT)
        pltpu.sync_copy(y_hbm.at[pl.ds(start, RPT)],
                        acc_sp.at[pl.ds(s * RPT, RPT)])

    @pl.when(s == 15)
    def _():
        start = pl.multiple_of(c * N + 15 * RPT, 16)
        pltpu.sync_copy(y_hbm.at[pl.ds(start, RPT_LAST)],
                        acc_sp.at[pl.ds(15 * RPT, RPT_LAST)])

    for k in (0, 1, 2):
        pltpu.async_copy(schunk_hbm(k), sidx[k], jsem[k])
    pltpu.async_copy(dchunk_hbm(0), didx0, isem0)
    pltpu.async_copy(dchunk_hbm(1), didx1, isem1)
    pltpu.make_async_copy(schunk_hbm(0), sidx0, jsem0).wait()
    pltpu.async_copy(y_hbm.at[sidx0], rows0, gsem0)
    pltpu.make_async_copy(schunk_hbm(1), sidx1, jsem1).wait()
    pltpu.async_copy(y_hbm.at[sidx1], rows1, gsem1)

    def step(t, _):
        for k in (0, 1, 2):
            j = 3 * t + k
            kp = (k + 2) % 3
            pltpu.make_async_copy(y_hbm.at[sidx[k]], rows[k], gsem[k]).wait()
            if k == 0:
                pltpu.async_copy(schunk_hbm(j + 3), sidx[k], jsem[k])
            else:
                @pl.when(t < NTRI - 1)
                def _(j=j, k=k):
                    pltpu.async_copy(schunk_hbm(j + 3), sidx[k], jsem[k])
            pltpu.make_async_copy(dchunk_hbm(j), didx[k], isem[k]).wait()
            pltpu.async_copy(rows[k], acc_sp.at[didx[k]], ssem[k], add=True)
            if k == 0:
                @pl.when(t > 0)
                def _():
                    pltpu.make_async_copy(rows2, acc_sp.at[didx2],
                                          ssem2).wait()
            else:
                pltpu.make_async_copy(rows[kp], acc_sp.at[didx[kp]],
                                      ssem[kp]).wait()
            if k == 2:
                @pl.when(t < NTRI - 1)
                def _(j=j, kp=kp):
                    pltpu.async_copy(dchunk_hbm(j + 2), didx[kp], isem[kp])
                    pltpu.make_async_copy(schunk_hbm(j + 2), sidx[kp],
                                          jsem[kp]).wait()
                    pltpu.async_copy(y_hbm.at[sidx[kp]], rows[kp], gsem[kp])
            else:
                pltpu.async_copy(dchunk_hbm(j + 2), didx[kp], isem[kp])
                pltpu.make_async_copy(schunk_hbm(j + 2), sidx[kp],
                                      jsem[kp]).wait()
                pltpu.async_copy(y_hbm.at[sidx[kp]], rows[kp], gsem[kp])
        return 0

    lax.fori_loop(0, NTRI, step, 0)
    pltpu.make_async_copy(y_hbm.at[sidx0], rows0, gsem0).wait()
    pltpu.make_async_copy(dchunk_hbm(NCHUNK - 1), didx0, isem0).wait()
    pltpu.sync_copy(rows0, acc_sp.at[didx0], add=True)
    pltpu.make_async_copy(rows2, acc_sp.at[didx2], ssem2).wait()
    plsc.subcore_barrier()

    @pl.when(s < 15)
    def _():
        pltpu.sync_copy(acc_sp.at[pl.ds(s * RPT, RPT)],
                        out_hbm.at[c, pl.ds(s * RPT, RPT)])

    @pl.when(s == 15)
    def _():
        pltpu.sync_copy(acc_sp.at[pl.ds(15 * RPT, RPT_LAST)],
                        out_hbm.at[c, pl.ds(15 * RPT, RPT_LAST)])


_agg_call = pl.kernel(
    _agg_body,
    out_type=jax.ShapeDtypeStruct((NC, N, 128), jnp.float32),
    mesh=_sc_mesh,
    scratch_types=(
        [pltpu.VMEM((K,), jnp.int32)] * 6
        + [pltpu.VMEM((K, 128), jnp.float32)] * 3
        + [pltpu.SemaphoreType.DMA] * 12
        + [pltpu.VMEM_SHARED((N, 128), jnp.float32)]
    ),
)


def _dinv(degp_ref):
    deg = degp_ref[0, :, 0:1] + degp_ref[1, :, 0:1] - 1.0
    return lax.rsqrt(deg)


def _split_out(y_ref, y):
    y_ref[0] = y[:, :128]
    y_ref[1] = y[:, 128:]


def _layer0_body(x_ref, degp_ref, w_ref, y_ref):
    y = _dinv(degp_ref) * jnp.dot(x_ref[...], w_ref[...],
                                  preferred_element_type=jnp.float32)
    _split_out(y_ref, y)


def _layer_body(acc_ref, degp_ref, b_ref, w_ref, y_ref):
    dinv = _dinv(degp_ref)
    acc = jnp.concatenate([acc_ref[0], acc_ref[1]], axis=1)
    h = jax.nn.relu(dinv * acc + b_ref[...])
    y = dinv * jnp.dot(h, w_ref[...], preferred_element_type=jnp.float32)
    _split_out(y_ref, y)


def _final_body(acc_ref, degp_ref, b_ref, wf1_ref, bf1_ref, wf2_ref, bf2_ref, o_ref):
    dinv = _dinv(degp_ref)
    acc = jnp.concatenate([acc_ref[0], acc_ref[1]], axis=1)
    h = jax.nn.relu(dinv * acc + b_ref[...])
    t = jnp.dot(h, wf1_ref[...], preferred_element_type=jnp.float32) + bf1_ref[...]
    o = jnp.dot(t, wf2_ref[...], preferred_element_type=jnp.float32) + bf2_ref[...]
    m = jnp.max(o, axis=1, keepdims=True)
    sh = o - m
    o_ref[...] = sh - jnp.log(jnp.sum(jnp.exp(sh), axis=1, keepdims=True))


def _row_spec(d):
    return pl.BlockSpec((ROW_BLK, d), lambda i: (i, 0))


def _split_spec(d):
    return pl.BlockSpec((NC, ROW_BLK, d), lambda i: (0, i, 0))


def _full_spec(a, b):
    return pl.BlockSpec((a, b), lambda i: (0, 0))


_GRID = (N // ROW_BLK,)


def _layer0(x, degp, W):
    return pl.pallas_call(
        _layer0_body,
        grid=_GRID,
        in_specs=[_row_spec(128), _split_spec(16), _full_spec(128, 256)],
        out_specs=_split_spec(128),
        out_shape=jax.ShapeDtypeStruct((NC, N, 128), jnp.float32),
    )(x, degp, W)


def _layer(acc, degp, b, W):
    return pl.pallas_call(
        _layer_body,
        grid=_GRID,
        in_specs=[_split_spec(128), _split_spec(16), _full_spec(1, 256),
                  _full_spec(256, 256)],
        out_specs=_split_spec(128),
        out_shape=jax.ShapeDtypeStruct((NC, N, 128), jnp.float32),
    )(acc, degp, b.reshape(1, -1), W)


def _final(acc, degp, b, Wf1, bf1, Wf2, bf2):
    return pl.pallas_call(
        _final_body,
        grid=_GRID,
        in_specs=[_split_spec(128), _split_spec(16), _full_spec(1, 256),
                  _full_spec(256, 256), _full_spec(1, 256),
                  _full_spec(256, 128), _full_spec(1, 128)],
        out_specs=_row_spec(128),
        out_shape=jax.ShapeDtypeStruct((N, 128), jnp.float32),
    )(acc, degp, b.reshape(1, -1), Wf1, bf1.reshape(1, -1), Wf2, bf2.reshape(1, -1))


def kernel(x, edge_index, W0, b0, W1, b1, W2, b2, Wf1, bf1, Wf2, bf2):
    src = edge_index[0].astype(jnp.int32)
    dst = edge_index[1].astype(jnp.int32)
    degp = _deg_call(dst, jnp.ones((N, 16), jnp.float32))
    srcoff = jnp.concatenate([src, src + N])

    def agg(y):
        return _agg_call(y.reshape(NC * N, 128), srcoff, dst)

    y = _layer0(x, degp, W0)
    acc = agg(y)
    y = _layer(acc, degp, b0, W1)
    acc = agg(y)
    y = _layer(acc, degp, b1, W2)
    acc = agg(y)
    return _final(acc, degp, b2, Wf1, bf1, Wf2, bf2)
